# baseline (speedup 1.0000x reference)
import functools

import jax
import jax.numpy as jnp
from jax import lax
from jax.experimental import pallas as pl
from jax.experimental.pallas import tpu as pltpu
from jax.experimental.pallas import tpu_sc as plsc

_BF = jnp.bfloat16
_F = jnp.float32

K_TOP = 64
M_Q = 16
HEADS = 4
HEAD_DIM = 16
_SENT = -3.0e38
_NT = 2
_G = 8


def _gelu(x):
    return 0.5 * x * (1.0 + lax.erf(x * 0.7071067811865476))


def _k1(x_ref, pooled_ref, wd_ref, wo_ref, bd_ref, bo_ref, act_ref, sc_ref):
    x = x_ref[0].astype(_BF)
    proj = jnp.dot(x, wd_ref[...], preferred_element_type=_F) + bd_ref[...]
    proj = _gelu(proj)
    act = jnp.dot(proj.astype(_BF), wo_ref[...], preferred_element_type=_F) + bo_ref[...]
    act_ref[0] = act.astype(_BF)
    ptxt = jnp.dot(pooled_ref[0].astype(_BF), wo_ref[...], preferred_element_type=_F) + bo_ref[...]
    w = (ptxt + 1e-8).astype(_BF)
    a2 = (act + 1e-8).astype(_BF)
    num = lax.dot_general(w, a2, (((1,), (1,)), ((), ())),
                          preferred_element_type=_F)
    nrm2 = lax.dot_general(jnp.ones((1,) + w.shape[1:], _BF), a2 * a2,
                           (((1,), (1,)), ((), ())),
                           preferred_element_type=_F)
    sc_ref[0] = num / jnp.maximum(jnp.sqrt(nrm2), 1e-12)


def _sc_topk_build(B, N):
    mesh = plsc.VectorSubcoreMesh(core_axis_name="c", subcore_axis_name="s")
    nchunk = N // 16

    @functools.partial(
        pl.kernel,
        out_type=jax.ShapeDtypeStruct((B * N,), jnp.int32),
        mesh=mesh,
        scratch_types=[
            pltpu.VMEM((N,), _F),
            pltpu.VMEM((N,), jnp.uint32),
            pltpu.VMEM((N,), jnp.int32),
        ],
        compiler_params=pltpu.CompilerParams(needs_layout_passes=False),
    )
    def sc_topk(sc_hbm, sel_hbm, s_v, m_v, sel_v):
        wid = lax.axis_index("s") * 2 + lax.axis_index("c")
        base = wid * N
        pltpu.sync_copy(sc_hbm.at[pl.ds(base, N)], s_v)

        def mapb(i, carry):
            u = plsc.bitcast(s_v[pl.ds(i * 16, 16)], jnp.uint32)
            neg = u >= jnp.uint32(0x80000000)
            m_v[pl.ds(i * 16, 16)] = jnp.where(
                neg, u ^ jnp.uint32(0xFFFFFFFF), u | jnp.uint32(0x80000000))
            return carry

        lax.fori_loop(0, nchunk, mapb, 0)

        def count_ge(th):
            def cb(i, acc):
                ge = m_v[pl.ds(i * 16, 16)] >= th
                return acc + plsc.all_reduce_population_count(ge)

            return lax.fori_loop(0, nchunk, cb, jnp.zeros((16,), jnp.int32))

        one = jnp.ones((16,), jnp.uint32)

        def bitb(j, acc):
            cand = acc | (one << (31 - j))
            return jnp.where(count_ge(cand) >= K_TOP, cand, acc)

        t = lax.fori_loop(0, 32, bitb, jnp.zeros((16,), jnp.uint32))
        budget = K_TOP - count_ge(t + 1)

        def comp(i, carry):
            nsel, tused = carry
            m = m_v[pl.ds(i * 16, 16)]
            gt = m > t
            eq = m == t
            ceq = plsc.cumsum(eq.astype(jnp.int32))
            take_eq = eq & ((tused + ceq) <= budget)
            selm = gt | take_eq
            csel = plsc.cumsum(selm.astype(jnp.int32))
            sel_v[pl.ds(i * 16, 16)] = jnp.where(selm, nsel + csel - 1, -1)
            return (nsel + plsc.all_reduce_population_count(selm),
                    tused + plsc.all_reduce_population_count(take_eq))

        z = jnp.zeros((16,), jnp.int32)
        lax.fori_loop(0, nchunk, comp, (z, z))
        pltpu.sync_copy(sel_v, sel_hbm.at[pl.ds(base, N)])

    return sc_topk


def _k34(act_ref, sel_ref, x_ref, mq_ref, lnw_ref, lnb_ref, wh_ref, bh_ref,
         woh_ref, bout_ref, wup_ref, bu_ref, g_ref, o_ref):
    gam = g_ref[0, 0]
    gbu = gam * bu_ref[...]
    gs = range(_G)
    n = sel_ref.shape[2]
    kio = lax.broadcasted_iota(jnp.int32, (K_TOP, n), 0)
    pmat = [(jnp.broadcast_to(sel_ref[g], (K_TOP, n)) == kio).astype(_BF)
            for g in gs]
    sparse = [jnp.dot(pmat[g], act_ref[g], preferred_element_type=_F) for g in gs]
    comb = [jnp.concatenate([mq_ref[...], sparse[g]], axis=0) for g in gs]
    mu = [jnp.mean(comb[g], axis=1, keepdims=True) for g in gs]
    var = [jnp.mean((comb[g] - mu[g]) ** 2, axis=1, keepdims=True) for g in gs]
    ln16 = [((comb[g] - mu[g]) * lax.rsqrt(var[g] + 1e-5) * lnw_ref[...]
             + lnb_ref[...]).astype(_BF) for g in gs]
    qh = [[jnp.dot(ln16[g], wh_ref[h], preferred_element_type=_F) + bh_ref[h]
           for h in range(HEADS)] for g in gs]
    kh = [[jnp.dot(ln16[g], wh_ref[HEADS + h], preferred_element_type=_F)
           + bh_ref[HEADS + h] for h in range(HEADS)] for g in gs]
    vh = [[jnp.dot(ln16[g], wh_ref[2 * HEADS + h], preferred_element_type=_F)
           + bh_ref[2 * HEADS + h] for h in range(HEADS)] for g in gs]
    lg = [[lax.dot_general(qh[g][h].astype(_BF), kh[g][h].astype(_BF),
                           (((1,), (1,)), ((), ())),
                           preferred_element_type=_F) * (1.0 / (HEAD_DIM ** 0.5))
           for h in range(HEADS)] for g in gs]
    mx = [[jnp.max(lg[g][h], axis=1, keepdims=True) for h in range(HEADS)]
          for g in gs]
    e = [[jnp.exp(lg[g][h] - mx[g][h]) for h in range(HEADS)] for g in gs]
    att = [[(e[g][h] / jnp.sum(e[g][h], axis=1, keepdims=True)).astype(_BF)
            for h in range(HEADS)] for g in gs]
    oh = [[jnp.dot(att[g][h], vh[g][h].astype(_BF), preferred_element_type=_F)
           for h in range(HEADS)] for g in gs]
    attn = [bout_ref[...] * jnp.ones((M_Q + K_TOP, 1), _F) for g in gs]
    for h in range(HEADS):
        attn = [attn[g] + jnp.dot(oh[g][h].astype(_BF), woh_ref[h],
                                  preferred_element_type=_F) for g in gs]
    enh = [comb[g] + attn[g] for g in gs]
    delta = [(jnp.dot(enh[g][M_Q:, :].astype(_BF), wup_ref[...],
                      preferred_element_type=_F) * gam).astype(_BF) for g in gs]
    scat = [lax.dot_general(pmat[g], delta[g], (((0,), (0,)), ((), ())),
                            preferred_element_type=_F) for g in gs]
    for g in gs:
        o_ref[g] = x_ref[g] + gbu + scat[g]


def kernel(image_features, text_features, W_down, b_down, W_omni, b_omni,
           W_up, b_up, m_queries, W_in, b_in, W_out, b_out, ln_w, ln_b, gamma):
    B, N, C = image_features.shape
    D = W_omni.shape[0]
    T = W_down.shape[0]
    pooled = text_features[:, None, 0, :]
    wd = W_down.T.astype(_BF)
    wo = W_omni.T.astype(_BF)
    bd = b_down.reshape(1, T)
    bo = b_omni.reshape(1, D)

    nt = N // _NT
    act, scores = pl.pallas_call(
        _k1,
        grid=(B, _NT),
        in_specs=[
            pl.BlockSpec((1, nt, C), lambda b, j: (b, j, 0)),
            pl.BlockSpec((1, 1, T), lambda b, j: (b, 0, 0)),
            pl.BlockSpec((C, T), lambda b, j: (0, 0)),
            pl.BlockSpec((T, D), lambda b, j: (0, 0)),
            pl.BlockSpec((1, T), lambda b, j: (0, 0)),
            pl.BlockSpec((1, D), lambda b, j: (0, 0)),
        ],
        out_specs=[
            pl.BlockSpec((1, nt, D), lambda b, j: (b, j, 0)),
            pl.BlockSpec((1, 1, nt), lambda b, j: (b, 0, j)),
        ],
        out_shape=[
            jax.ShapeDtypeStruct((B, N, D), _BF),
            jax.ShapeDtypeStruct((B, 1, N), _F),
        ],
        interpret=False,
    )(image_features, pooled, wd, wo, bd, bo)

    sel = _sc_topk_build(B, N)(scores.reshape(B * N))
    sel3 = sel.reshape(B, 1, N)

    wq = W_in[:D].T
    wk = W_in[D:2 * D].T
    wv = W_in[2 * D:].T
    wh = jnp.stack(
        [wq[:, h * HEAD_DIM:(h + 1) * HEAD_DIM] for h in range(HEADS)]
        + [wk[:, h * HEAD_DIM:(h + 1) * HEAD_DIM] for h in range(HEADS)]
        + [wv[:, h * HEAD_DIM:(h + 1) * HEAD_DIM] for h in range(HEADS)]
    ).astype(_BF)
    bh = jnp.stack([b_in[i * HEAD_DIM:(i + 1) * HEAD_DIM].reshape(1, HEAD_DIM)
                    for i in range(3 * HEADS)])
    wot = W_out.T
    woh = jnp.stack([wot[h * HEAD_DIM:(h + 1) * HEAD_DIM, :]
                     for h in range(HEADS)]).astype(_BF)
    mq = m_queries[0]
    lnw2 = ln_w.reshape(1, D)
    lnb2 = ln_b.reshape(1, D)
    bout2 = b_out.reshape(1, D)
    wup = W_up.T.astype(_BF)
    bu2 = b_up.reshape(1, C)
    g2 = jnp.reshape(gamma, (1, 1)).astype(_F)

    out = pl.pallas_call(
        _k34,
        grid=(B // _G,),
        in_specs=[
            pl.BlockSpec((_G, N, D), lambda b: (b, 0, 0)),
            pl.BlockSpec((_G, 1, N), lambda b: (b, 0, 0)),
            pl.BlockSpec((_G, N, C), lambda b: (b, 0, 0)),
            pl.BlockSpec((M_Q, D), lambda b: (0, 0)),
            pl.BlockSpec((1, D), lambda b: (0, 0)),
            pl.BlockSpec((1, D), lambda b: (0, 0)),
            pl.BlockSpec((3 * HEADS, D, HEAD_DIM), lambda b: (0, 0, 0)),
            pl.BlockSpec((3 * HEADS, 1, HEAD_DIM), lambda b: (0, 0, 0)),
            pl.BlockSpec((HEADS, HEAD_DIM, D), lambda b: (0, 0, 0)),
            pl.BlockSpec((1, D), lambda b: (0, 0)),
            pl.BlockSpec((D, C), lambda b: (0, 0)),
            pl.BlockSpec((1, C), lambda b: (0, 0)),
            pl.BlockSpec((1, 1), lambda b: (0, 0)),
        ],
        out_specs=pl.BlockSpec((_G, N, C), lambda b: (b, 0, 0)),
        out_shape=jax.ShapeDtypeStruct((B, N, C), _F),
        interpret=False,
    )(act, sel3, image_features, mq, lnw2, lnb2, wh, bh, woh, bout2, wup,
      bu2, g2)
    return out


# speedup vs baseline: 1.1455x; 1.1455x over previous
import functools

import jax
import jax.numpy as jnp
from jax import lax
from jax.experimental import pallas as pl
from jax.experimental.pallas import tpu as pltpu
from jax.experimental.pallas import tpu_sc as plsc

_BF = jnp.bfloat16
_F = jnp.float32

K_TOP = 64
M_Q = 16
HEADS = 4
HEAD_DIM = 16
_SENT = -3.0e38
_NT = 1
_G = 8


def _gelu(x):
    return 0.5 * x * (1.0 + lax.erf(x * 0.7071067811865476))


def _k1(x_ref, pooled_ref, wd_ref, wo_ref, bd_ref, bo_ref, sc_ref):
    x = x_ref[0].astype(_BF)
    proj = jnp.dot(x, wd_ref[...], preferred_element_type=_F) + bd_ref[...]
    proj = _gelu(proj)
    act = jnp.dot(proj.astype(_BF), wo_ref[...], preferred_element_type=_F) + bo_ref[...]
    ptxt = jnp.dot(pooled_ref[0].astype(_BF), wo_ref[...], preferred_element_type=_F) + bo_ref[...]
    w = (ptxt + 1e-8).astype(_BF)
    a2 = (act + 1e-8).astype(_BF)
    num = lax.dot_general(w, a2, (((1,), (1,)), ((), ())),
                          preferred_element_type=_F)
    nrm2 = lax.dot_general(jnp.ones((1,) + w.shape[1:], _BF), a2 * a2,
                           (((1,), (1,)), ((), ())),
                           preferred_element_type=_F)
    sc_ref[0] = num / jnp.maximum(jnp.sqrt(nrm2), 1e-12)


def _sc_topk_build(B, N):
    mesh = plsc.VectorSubcoreMesh(core_axis_name="c", subcore_axis_name="s")
    nchunk = N // 16

    @functools.partial(
        pl.kernel,
        out_type=jax.ShapeDtypeStruct((B * N,), jnp.int32),
        mesh=mesh,
        scratch_types=[
            pltpu.VMEM((N,), _F),
            pltpu.VMEM((N,), jnp.uint32),
            pltpu.VMEM((N,), jnp.int32),
        ],
        compiler_params=pltpu.CompilerParams(needs_layout_passes=False),
    )
    def sc_topk(sc_hbm, sel_hbm, s_v, m_v, sel_v):
        wid = lax.axis_index("s") * 2 + lax.axis_index("c")
        base = wid * N
        pltpu.sync_copy(sc_hbm.at[pl.ds(base, N)], s_v)

        def mapb(i, carry):
            u = plsc.bitcast(s_v[pl.ds(i * 16, 16)], jnp.uint32)
            neg = u >= jnp.uint32(0x80000000)
            m_v[pl.ds(i * 16, 16)] = jnp.where(
                neg, u ^ jnp.uint32(0xFFFFFFFF), u | jnp.uint32(0x80000000))
            return carry

        lax.fori_loop(0, nchunk, mapb, 0)

        def count_ge(th):
            def cb(i, acc):
                ge = m_v[pl.ds(i * 16, 16)] >= th
                return acc + plsc.all_reduce_population_count(ge)

            return lax.fori_loop(0, nchunk, cb, jnp.zeros((16,), jnp.int32))

        one = jnp.ones((16,), jnp.uint32)

        def bitb(j, acc):
            cand = acc | (one << (31 - j))
            return jnp.where(count_ge(cand) >= K_TOP, cand, acc)

        t = lax.fori_loop(0, 32, bitb, jnp.zeros((16,), jnp.uint32))
        budget = K_TOP - count_ge(t + 1)

        def comp(i, carry):
            nsel, tused = carry
            m = m_v[pl.ds(i * 16, 16)]
            gt = m > t
            eq = m == t
            ceq = plsc.cumsum(eq.astype(jnp.int32))
            take_eq = eq & ((tused + ceq) <= budget)
            selm = gt | take_eq
            csel = plsc.cumsum(selm.astype(jnp.int32))
            sel_v[pl.ds(i * 16, 16)] = jnp.where(selm, nsel + csel - 1, -1)
            return (nsel + plsc.all_reduce_population_count(selm),
                    tused + plsc.all_reduce_population_count(take_eq))

        z = jnp.zeros((16,), jnp.int32)
        lax.fori_loop(0, nchunk, comp, (z, z))
        pltpu.sync_copy(sel_v, sel_hbm.at[pl.ds(base, N)])

    return sc_topk


def _k34(sel_ref, x_ref, wd_ref, wo_ref, bd_ref, bo_ref, mq_ref, lnw_ref,
         lnb_ref, wh_ref, bh_ref, woh_ref, bout_ref, wup_ref, bu_ref, g_ref,
         o_ref):
    gam = g_ref[0, 0]
    gbu = gam * bu_ref[...]
    gs = range(_G)
    n = sel_ref.shape[2]
    kio = lax.broadcasted_iota(jnp.int32, (K_TOP, n), 0)
    pmat = [(jnp.broadcast_to(sel_ref[g], (K_TOP, n)) == kio).astype(_BF)
            for g in gs]
    xg = [jnp.dot(pmat[g], x_ref[g].astype(_BF), preferred_element_type=_F)
          for g in gs]
    pj = [_gelu(jnp.dot(xg[g].astype(_BF), wd_ref[...],
                        preferred_element_type=_F) + bd_ref[...]) for g in gs]
    sparse = [jnp.dot(pj[g].astype(_BF), wo_ref[...],
                      preferred_element_type=_F) + bo_ref[...] for g in gs]
    comb = [jnp.concatenate([mq_ref[...], sparse[g]], axis=0) for g in gs]
    mu = [jnp.mean(comb[g], axis=1, keepdims=True) for g in gs]
    var = [jnp.mean((comb[g] - mu[g]) ** 2, axis=1, keepdims=True) for g in gs]
    ln16 = [((comb[g] - mu[g]) * lax.rsqrt(var[g] + 1e-5) * lnw_ref[...]
             + lnb_ref[...]).astype(_BF) for g in gs]
    qh = [[jnp.dot(ln16[g], wh_ref[h], preferred_element_type=_F) + bh_ref[h]
           for h in range(HEADS)] for g in gs]
    kh = [[jnp.dot(ln16[g], wh_ref[HEADS + h], preferred_element_type=_F)
           + bh_ref[HEADS + h] for h in range(HEADS)] for g in gs]
    vh = [[jnp.dot(ln16[g], wh_ref[2 * HEADS + h], preferred_element_type=_F)
           + bh_ref[2 * HEADS + h] for h in range(HEADS)] for g in gs]
    lg = [[lax.dot_general(qh[g][h].astype(_BF), kh[g][h].astype(_BF),
                           (((1,), (1,)), ((), ())),
                           preferred_element_type=_F) * (1.0 / (HEAD_DIM ** 0.5))
           for h in range(HEADS)] for g in gs]
    mx = [[jnp.max(lg[g][h], axis=1, keepdims=True) for h in range(HEADS)]
          for g in gs]
    e = [[jnp.exp(lg[g][h] - mx[g][h]) for h in range(HEADS)] for g in gs]
    att = [[(e[g][h] / jnp.sum(e[g][h], axis=1, keepdims=True)).astype(_BF)
            for h in range(HEADS)] for g in gs]
    oh = [[jnp.dot(att[g][h], vh[g][h].astype(_BF), preferred_element_type=_F)
           for h in range(HEADS)] for g in gs]
    attn = [bout_ref[...] * jnp.ones((M_Q + K_TOP, 1), _F) for g in gs]
    for h in range(HEADS):
        attn = [attn[g] + jnp.dot(oh[g][h].astype(_BF), woh_ref[h],
                                  preferred_element_type=_F) for g in gs]
    enh = [comb[g] + attn[g] for g in gs]
    delta = [(jnp.dot(enh[g][M_Q:, :].astype(_BF), wup_ref[...],
                      preferred_element_type=_F) * gam).astype(_BF) for g in gs]
    scat = [lax.dot_general(pmat[g], delta[g], (((0,), (0,)), ((), ())),
                            preferred_element_type=_F) for g in gs]
    for g in gs:
        o_ref[g] = x_ref[g] + gbu + scat[g]


def kernel(image_features, text_features, W_down, b_down, W_omni, b_omni,
           W_up, b_up, m_queries, W_in, b_in, W_out, b_out, ln_w, ln_b, gamma):
    B, N, C = image_features.shape
    D = W_omni.shape[0]
    T = W_down.shape[0]
    pooled = text_features[:, None, 0, :]
    wd = W_down.T.astype(_BF)
    wo = W_omni.T.astype(_BF)
    bd = b_down.reshape(1, T)
    bo = b_omni.reshape(1, D)

    nt = N // _NT
    scores = pl.pallas_call(
        _k1,
        grid=(B, _NT),
        in_specs=[
            pl.BlockSpec((1, nt, C), lambda b, j: (b, j, 0)),
            pl.BlockSpec((1, 1, T), lambda b, j: (b, 0, 0)),
            pl.BlockSpec((C, T), lambda b, j: (0, 0)),
            pl.BlockSpec((T, D), lambda b, j: (0, 0)),
            pl.BlockSpec((1, T), lambda b, j: (0, 0)),
            pl.BlockSpec((1, D), lambda b, j: (0, 0)),
        ],
        out_specs=pl.BlockSpec((1, 1, nt), lambda b, j: (b, 0, j)),
        out_shape=jax.ShapeDtypeStruct((B, 1, N), _F),
        interpret=False,
    )(image_features, pooled, wd, wo, bd, bo)

    sel = _sc_topk_build(B, N)(scores.reshape(B * N))
    sel3 = sel.reshape(B, 1, N)

    wq = W_in[:D].T
    wk = W_in[D:2 * D].T
    wv = W_in[2 * D:].T
    wh = jnp.stack(
        [wq[:, h * HEAD_DIM:(h + 1) * HEAD_DIM] for h in range(HEADS)]
        + [wk[:, h * HEAD_DIM:(h + 1) * HEAD_DIM] for h in range(HEADS)]
        + [wv[:, h * HEAD_DIM:(h + 1) * HEAD_DIM] for h in range(HEADS)]
    ).astype(_BF)
    bh = jnp.stack([b_in[i * HEAD_DIM:(i + 1) * HEAD_DIM].reshape(1, HEAD_DIM)
                    for i in range(3 * HEADS)])
    wot = W_out.T
    woh = jnp.stack([wot[h * HEAD_DIM:(h + 1) * HEAD_DIM, :]
                     for h in range(HEADS)]).astype(_BF)
    mq = m_queries[0]
    lnw2 = ln_w.reshape(1, D)
    lnb2 = ln_b.reshape(1, D)
    bout2 = b_out.reshape(1, D)
    wup = W_up.T.astype(_BF)
    bu2 = b_up.reshape(1, C)
    g2 = jnp.reshape(gamma, (1, 1)).astype(_F)

    out = pl.pallas_call(
        _k34,
        grid=(B // _G,),
        in_specs=[
            pl.BlockSpec((_G, 1, N), lambda b: (b, 0, 0)),
            pl.BlockSpec((_G, N, C), lambda b: (b, 0, 0)),
            pl.BlockSpec((C, T), lambda b: (0, 0)),
            pl.BlockSpec((T, D), lambda b: (0, 0)),
            pl.BlockSpec((1, T), lambda b: (0, 0)),
            pl.BlockSpec((1, D), lambda b: (0, 0)),
            pl.BlockSpec((M_Q, D), lambda b: (0, 0)),
            pl.BlockSpec((1, D), lambda b: (0, 0)),
            pl.BlockSpec((1, D), lambda b: (0, 0)),
            pl.BlockSpec((3 * HEADS, D, HEAD_DIM), lambda b: (0, 0, 0)),
            pl.BlockSpec((3 * HEADS, 1, HEAD_DIM), lambda b: (0, 0, 0)),
            pl.BlockSpec((HEADS, HEAD_DIM, D), lambda b: (0, 0, 0)),
            pl.BlockSpec((1, D), lambda b: (0, 0)),
            pl.BlockSpec((D, C), lambda b: (0, 0)),
            pl.BlockSpec((1, C), lambda b: (0, 0)),
            pl.BlockSpec((1, 1), lambda b: (0, 0)),
        ],
        out_specs=pl.BlockSpec((_G, N, C), lambda b: (b, 0, 0)),
        out_shape=jax.ShapeDtypeStruct((B, N, C), _F),
        interpret=False,
    )(sel3, image_features, wd, wo, bd, bo, mq, lnw2, lnb2, wh, bh, woh,
      bout2, wup, bu2, g2)
    return out


# speedup vs baseline: 1.1963x; 1.0443x over previous
import functools

import jax
import jax.numpy as jnp
from jax import lax
from jax.experimental import pallas as pl
from jax.experimental.pallas import tpu as pltpu
from jax.experimental.pallas import tpu_sc as plsc

_BF = jnp.bfloat16
_F = jnp.float32

K_TOP = 64
M_Q = 16
HEADS = 4
HEAD_DIM = 16
_SENT = -3.0e38
_KB = 2
_G = 8


def _gelu(x):
    return 0.5 * x * (1.0 + lax.erf(x * 0.7071067811865476))


def _k1(x_ref, pooled_ref, wd_ref, wo_ref, bd_ref, bo_ref, sc_ref):
    kb, n = x_ref.shape[0], x_ref.shape[1]
    x = x_ref[...].reshape(kb * n, -1).astype(_BF)
    proj = jnp.dot(x, wd_ref[...], preferred_element_type=_F) + bd_ref[...]
    proj = _gelu(proj)
    act = jnp.dot(proj.astype(_BF), wo_ref[...], preferred_element_type=_F) + bo_ref[...]
    ptxt = jnp.dot(pooled_ref[:, 0, :].astype(_BF), wo_ref[...],
                   preferred_element_type=_F) + bo_ref[...]
    w = (ptxt + 1e-8).astype(_BF)
    a2 = (act + 1e-8).astype(_BF)
    ones = jnp.ones((1,) + w.shape[1:], _BF)
    for i in range(kb):
        a2i = a2[i * n:(i + 1) * n, :]
        num = lax.dot_general(w[i:i + 1, :], a2i, (((1,), (1,)), ((), ())),
                              preferred_element_type=_F)
        nrm2 = lax.dot_general(ones, a2i * a2i, (((1,), (1,)), ((), ())),
                               preferred_element_type=_F)
        sc_ref[i] = num / jnp.maximum(jnp.sqrt(nrm2), 1e-12)


def _sc_topk_build(B, N):
    mesh = plsc.VectorSubcoreMesh(core_axis_name="c", subcore_axis_name="s")
    nchunk = N // 16

    @functools.partial(
        pl.kernel,
        out_type=jax.ShapeDtypeStruct((B * N,), jnp.int32),
        mesh=mesh,
        scratch_types=[
            pltpu.VMEM((N,), _F),
            pltpu.VMEM((N,), jnp.uint32),
            pltpu.VMEM((N,), jnp.int32),
        ],
        compiler_params=pltpu.CompilerParams(needs_layout_passes=False),
    )
    def sc_topk(sc_hbm, sel_hbm, s_v, m_v, sel_v):
        wid = lax.axis_index("s") * 2 + lax.axis_index("c")
        base = wid * N
        pltpu.sync_copy(sc_hbm.at[pl.ds(base, N)], s_v)

        def mapb(i, carry):
            u = plsc.bitcast(s_v[pl.ds(i * 16, 16)], jnp.uint32)
            neg = u >= jnp.uint32(0x80000000)
            m_v[pl.ds(i * 16, 16)] = jnp.where(
                neg, u ^ jnp.uint32(0xFFFFFFFF), u | jnp.uint32(0x80000000))
            return carry

        lax.fori_loop(0, nchunk, mapb, 0)

        def count_ge(th):
            def cb(i, acc):
                ge = m_v[pl.ds(i * 16, 16)] >= th
                return acc + plsc.all_reduce_population_count(ge)

            return lax.fori_loop(0, nchunk, cb, jnp.zeros((16,), jnp.int32))

        one = jnp.ones((16,), jnp.uint32)

        def bitb(j, acc):
            cand = acc | (one << (31 - j))
            return jnp.where(count_ge(cand) >= K_TOP, cand, acc)

        t = lax.fori_loop(0, 32, bitb, jnp.zeros((16,), jnp.uint32))
        budget = K_TOP - count_ge(t + 1)

        def comp(i, carry):
            nsel, tused = carry
            m = m_v[pl.ds(i * 16, 16)]
            gt = m > t
            eq = m == t
            ceq = plsc.cumsum(eq.astype(jnp.int32))
            take_eq = eq & ((tused + ceq) <= budget)
            selm = gt | take_eq
            csel = plsc.cumsum(selm.astype(jnp.int32))
            sel_v[pl.ds(i * 16, 16)] = jnp.where(selm, nsel + csel - 1, -1)
            return (nsel + plsc.all_reduce_population_count(selm),
                    tused + plsc.all_reduce_population_count(take_eq))

        z = jnp.zeros((16,), jnp.int32)
        lax.fori_loop(0, nchunk, comp, (z, z))
        pltpu.sync_copy(sel_v, sel_hbm.at[pl.ds(base, N)])

    return sc_topk


def _k34(sel_ref, x_ref, wd_ref, wo_ref, bd_ref, bo_ref, mq_ref, lnw_ref,
         lnb_ref, wh_ref, bh_ref, woh_ref, bout_ref, wup_ref, bu_ref, g_ref,
         o_ref):
    gam = g_ref[0, 0]
    gbu = gam * bu_ref[...]
    gs = range(_G)
    n = sel_ref.shape[2]
    kio = lax.broadcasted_iota(jnp.int32, (K_TOP, n), 0)
    pmat = [(jnp.broadcast_to(sel_ref[g], (K_TOP, n)) == kio).astype(_BF)
            for g in gs]
    xg = [jnp.dot(pmat[g], x_ref[g].astype(_BF), preferred_element_type=_F)
          for g in gs]
    pj = [_gelu(jnp.dot(xg[g].astype(_BF), wd_ref[...],
                        preferred_element_type=_F) + bd_ref[...]) for g in gs]
    sparse = [jnp.dot(pj[g].astype(_BF), wo_ref[...],
                      preferred_element_type=_F) + bo_ref[...] for g in gs]
    comb = [jnp.concatenate([mq_ref[...], sparse[g]], axis=0) for g in gs]
    mu = [jnp.mean(comb[g], axis=1, keepdims=True) for g in gs]
    var = [jnp.mean((comb[g] - mu[g]) ** 2, axis=1, keepdims=True) for g in gs]
    ln16 = [((comb[g] - mu[g]) * lax.rsqrt(var[g] + 1e-5) * lnw_ref[...]
             + lnb_ref[...]).astype(_BF) for g in gs]
    qh = [[jnp.dot(ln16[g], wh_ref[h], preferred_element_type=_F) + bh_ref[h]
           for h in range(HEADS)] for g in gs]
    kh = [[jnp.dot(ln16[g], wh_ref[HEADS + h], preferred_element_type=_F)
           + bh_ref[HEADS + h] for h in range(HEADS)] for g in gs]
    vh = [[jnp.dot(ln16[g], wh_ref[2 * HEADS + h], preferred_element_type=_F)
           + bh_ref[2 * HEADS + h] for h in range(HEADS)] for g in gs]
    lg = [[lax.dot_general(qh[g][h].astype(_BF), kh[g][h].astype(_BF),
                           (((1,), (1,)), ((), ())),
                           preferred_element_type=_F) * (1.0 / (HEAD_DIM ** 0.5))
           for h in range(HEADS)] for g in gs]
    mx = [[jnp.max(lg[g][h], axis=1, keepdims=True) for h in range(HEADS)]
          for g in gs]
    e = [[jnp.exp(lg[g][h] - mx[g][h]) for h in range(HEADS)] for g in gs]
    att = [[(e[g][h] / jnp.sum(e[g][h], axis=1, keepdims=True)).astype(_BF)
            for h in range(HEADS)] for g in gs]
    oh = [[jnp.dot(att[g][h], vh[g][h].astype(_BF), preferred_element_type=_F)
           for h in range(HEADS)] for g in gs]
    attn = [bout_ref[...] * jnp.ones((M_Q + K_TOP, 1), _F) for g in gs]
    for h in range(HEADS):
        attn = [attn[g] + jnp.dot(oh[g][h].astype(_BF), woh_ref[h],
                                  preferred_element_type=_F) for g in gs]
    enh = [comb[g] + attn[g] for g in gs]
    delta = [(jnp.dot(enh[g][M_Q:, :].astype(_BF), wup_ref[...],
                      preferred_element_type=_F) * gam).astype(_BF) for g in gs]
    scat = [lax.dot_general(pmat[g], delta[g], (((0,), (0,)), ((), ())),
                            preferred_element_type=_F) for g in gs]
    for g in gs:
        o_ref[g] = x_ref[g] + gbu + scat[g]


def kernel(image_features, text_features, W_down, b_down, W_omni, b_omni,
           W_up, b_up, m_queries, W_in, b_in, W_out, b_out, ln_w, ln_b, gamma):
    B, N, C = image_features.shape
    D = W_omni.shape[0]
    T = W_down.shape[0]
    pooled = text_features[:, None, 0, :]
    wd = W_down.T.astype(_BF)
    wo = W_omni.T.astype(_BF)
    bd = b_down.reshape(1, T)
    bo = b_omni.reshape(1, D)

    scores = pl.pallas_call(
        _k1,
        grid=(B // _KB,),
        in_specs=[
            pl.BlockSpec((_KB, N, C), lambda b: (b, 0, 0)),
            pl.BlockSpec((_KB, 1, T), lambda b: (b, 0, 0)),
            pl.BlockSpec((C, T), lambda b: (0, 0)),
            pl.BlockSpec((T, D), lambda b: (0, 0)),
            pl.BlockSpec((1, T), lambda b: (0, 0)),
            pl.BlockSpec((1, D), lambda b: (0, 0)),
        ],
        out_specs=pl.BlockSpec((_KB, 1, N), lambda b: (b, 0, 0)),
        out_shape=jax.ShapeDtypeStruct((B, 1, N), _F),
        interpret=False,
    )(image_features, pooled, wd, wo, bd, bo)

    sel = _sc_topk_build(B, N)(scores.reshape(B * N))
    sel3 = sel.reshape(B, 1, N)

    wq = W_in[:D].T
    wk = W_in[D:2 * D].T
    wv = W_in[2 * D:].T
    wh = jnp.stack(
        [wq[:, h * HEAD_DIM:(h + 1) * HEAD_DIM] for h in range(HEADS)]
        + [wk[:, h * HEAD_DIM:(h + 1) * HEAD_DIM] for h in range(HEADS)]
        + [wv[:, h * HEAD_DIM:(h + 1) * HEAD_DIM] for h in range(HEADS)]
    ).astype(_BF)
    bh = jnp.stack([b_in[i * HEAD_DIM:(i + 1) * HEAD_DIM].reshape(1, HEAD_DIM)
                    for i in range(3 * HEADS)])
    wot = W_out.T
    woh = jnp.stack([wot[h * HEAD_DIM:(h + 1) * HEAD_DIM, :]
                     for h in range(HEADS)]).astype(_BF)
    mq = m_queries[0]
    lnw2 = ln_w.reshape(1, D)
    lnb2 = ln_b.reshape(1, D)
    bout2 = b_out.reshape(1, D)
    wup = W_up.T.astype(_BF)
    bu2 = b_up.reshape(1, C)
    g2 = jnp.reshape(gamma, (1, 1)).astype(_F)

    out = pl.pallas_call(
        _k34,
        grid=(B // _G,),
        in_specs=[
            pl.BlockSpec((_G, 1, N), lambda b: (b, 0, 0)),
            pl.BlockSpec((_G, N, C), lambda b: (b, 0, 0)),
            pl.BlockSpec((C, T), lambda b: (0, 0)),
            pl.BlockSpec((T, D), lambda b: (0, 0)),
            pl.BlockSpec((1, T), lambda b: (0, 0)),
            pl.BlockSpec((1, D), lambda b: (0, 0)),
            pl.BlockSpec((M_Q, D), lambda b: (0, 0)),
            pl.BlockSpec((1, D), lambda b: (0, 0)),
            pl.BlockSpec((1, D), lambda b: (0, 0)),
            pl.BlockSpec((3 * HEADS, D, HEAD_DIM), lambda b: (0, 0, 0)),
            pl.BlockSpec((3 * HEADS, 1, HEAD_DIM), lambda b: (0, 0, 0)),
            pl.BlockSpec((HEADS, HEAD_DIM, D), lambda b: (0, 0, 0)),
            pl.BlockSpec((1, D), lambda b: (0, 0)),
            pl.BlockSpec((D, C), lambda b: (0, 0)),
            pl.BlockSpec((1, C), lambda b: (0, 0)),
            pl.BlockSpec((1, 1), lambda b: (0, 0)),
        ],
        out_specs=pl.BlockSpec((_G, N, C), lambda b: (b, 0, 0)),
        out_shape=jax.ShapeDtypeStruct((B, N, C), _F),
        interpret=False,
    )(sel3, image_features, wd, wo, bd, bo, mq, lnw2, lnb2, wh, bh, woh,
      bout2, wup, bu2, g2)
    return out


# speedup vs baseline: 1.2116x; 1.0128x over previous
import functools

import jax
import jax.numpy as jnp
from jax import lax
from jax.experimental import pallas as pl
from jax.experimental.pallas import tpu as pltpu
from jax.experimental.pallas import tpu_sc as plsc

_BF = jnp.bfloat16
_F = jnp.float32

K_TOP = 64
M_Q = 16
HEADS = 4
HEAD_DIM = 16
_SENT = -3.0e38
_KB = 4
_G = 8


def _gelu(x):
    return 0.5 * x * (1.0 + lax.erf(x * 0.7071067811865476))


def _k1(x_ref, pooled_ref, wd_ref, wo_ref, bd_ref, bo_ref, sc_ref):
    kb, n = x_ref.shape[0], x_ref.shape[1]
    x = x_ref[...].reshape(kb * n, -1).astype(_BF)
    proj = jnp.dot(x, wd_ref[...], preferred_element_type=_F) + bd_ref[...]
    proj = _gelu(proj)
    act = jnp.dot(proj.astype(_BF), wo_ref[...], preferred_element_type=_F) + bo_ref[...]
    ptxt = jnp.dot(pooled_ref[:, 0, :].astype(_BF), wo_ref[...],
                   preferred_element_type=_F) + bo_ref[...]
    w = (ptxt + 1e-8).astype(_BF)
    a2 = (act + 1e-8).astype(_BF)
    ones = jnp.ones((1,) + w.shape[1:], _BF)
    for i in range(kb):
        a2i = a2[i * n:(i + 1) * n, :]
        num = lax.dot_general(w[i:i + 1, :], a2i, (((1,), (1,)), ((), ())),
                              preferred_element_type=_F)
        nrm2 = lax.dot_general(ones, a2i * a2i, (((1,), (1,)), ((), ())),
                               preferred_element_type=_F)
        sc_ref[i] = num / jnp.maximum(jnp.sqrt(nrm2), 1e-12)


def _sc_topk_build(B, N):
    mesh = plsc.VectorSubcoreMesh(core_axis_name="c", subcore_axis_name="s")
    nchunk = N // 16

    @functools.partial(
        pl.kernel,
        out_type=jax.ShapeDtypeStruct((B * N,), jnp.int32),
        mesh=mesh,
        scratch_types=[
            pltpu.VMEM((N,), _F),
            pltpu.VMEM((N,), jnp.uint32),
            pltpu.VMEM((N,), jnp.int32),
        ],
        compiler_params=pltpu.CompilerParams(needs_layout_passes=False),
    )
    def sc_topk(sc_hbm, sel_hbm, s_v, m_v, sel_v):
        wid = lax.axis_index("s") * 2 + lax.axis_index("c")
        base = wid * N
        pltpu.sync_copy(sc_hbm.at[pl.ds(base, N)], s_v)

        def mapb(i, carry):
            u = plsc.bitcast(s_v[pl.ds(i * 16, 16)], jnp.uint32)
            neg = u >= jnp.uint32(0x80000000)
            m_v[pl.ds(i * 16, 16)] = jnp.where(
                neg, u ^ jnp.uint32(0xFFFFFFFF), u | jnp.uint32(0x80000000))
            return carry

        lax.fori_loop(0, nchunk, mapb, 0)

        def count_ge(th):
            def cb(i, acc):
                ge = m_v[pl.ds(i * 16, 16)] >= th
                return acc + plsc.all_reduce_population_count(ge)

            return lax.fori_loop(0, nchunk, cb, jnp.zeros((16,), jnp.int32))

        one = jnp.ones((16,), jnp.uint32)

        def bitb(j, acc):
            cand = acc | (one << (31 - j))
            return jnp.where(count_ge(cand) >= K_TOP, cand, acc)

        t = lax.fori_loop(0, 32, bitb, jnp.zeros((16,), jnp.uint32))
        budget = K_TOP - count_ge(t + 1)

        def comp(i, carry):
            nsel, tused = carry
            m = m_v[pl.ds(i * 16, 16)]
            gt = m > t
            eq = m == t
            ceq = plsc.cumsum(eq.astype(jnp.int32))
            take_eq = eq & ((tused + ceq) <= budget)
            selm = gt | take_eq
            csel = plsc.cumsum(selm.astype(jnp.int32))
            sel_v[pl.ds(i * 16, 16)] = jnp.where(selm, nsel + csel - 1, -1)
            return (nsel + plsc.all_reduce_population_count(selm),
                    tused + plsc.all_reduce_population_count(take_eq))

        z = jnp.zeros((16,), jnp.int32)
        lax.fori_loop(0, nchunk, comp, (z, z))
        pltpu.sync_copy(sel_v, sel_hbm.at[pl.ds(base, N)])

    return sc_topk


def _k34(sel_ref, x_ref, wd_ref, wo_ref, bd_ref, bo_ref, mq_ref, lnw_ref,
         lnb_ref, wh_ref, bh_ref, woh_ref, bout_ref, wup_ref, bu_ref, g_ref,
         o_ref):
    gam = g_ref[0, 0]
    gbu = gam * bu_ref[...]
    gs = range(_G)
    n = sel_ref.shape[2]
    kio = lax.broadcasted_iota(jnp.int32, (K_TOP, n), 0)
    pmat = [(jnp.broadcast_to(sel_ref[g], (K_TOP, n)) == kio).astype(_BF)
            for g in gs]
    xg = [jnp.dot(pmat[g], x_ref[g].astype(_BF), preferred_element_type=_F)
          for g in gs]
    pj = [_gelu(jnp.dot(xg[g].astype(_BF), wd_ref[...],
                        preferred_element_type=_F) + bd_ref[...]) for g in gs]
    sparse = [jnp.dot(pj[g].astype(_BF), wo_ref[...],
                      preferred_element_type=_F) + bo_ref[...] for g in gs]
    comb = [jnp.concatenate([mq_ref[...], sparse[g]], axis=0) for g in gs]
    mu = [jnp.mean(comb[g], axis=1, keepdims=True) for g in gs]
    var = [jnp.mean((comb[g] - mu[g]) ** 2, axis=1, keepdims=True) for g in gs]
    ln16 = [((comb[g] - mu[g]) * lax.rsqrt(var[g] + 1e-5) * lnw_ref[...]
             + lnb_ref[...]).astype(_BF) for g in gs]
    qh = [[jnp.dot(ln16[g], wh_ref[h], preferred_element_type=_F) + bh_ref[h]
           for h in range(HEADS)] for g in gs]
    kh = [[jnp.dot(ln16[g], wh_ref[HEADS + h], preferred_element_type=_F)
           + bh_ref[HEADS + h] for h in range(HEADS)] for g in gs]
    vh = [[jnp.dot(ln16[g], wh_ref[2 * HEADS + h], preferred_element_type=_F)
           + bh_ref[2 * HEADS + h] for h in range(HEADS)] for g in gs]
    lg = [[lax.dot_general(qh[g][h].astype(_BF), kh[g][h].astype(_BF),
                           (((1,), (1,)), ((), ())),
                           preferred_element_type=_F) * (1.0 / (HEAD_DIM ** 0.5))
           for h in range(HEADS)] for g in gs]
    mx = [[jnp.max(lg[g][h], axis=1, keepdims=True) for h in range(HEADS)]
          for g in gs]
    e = [[jnp.exp(lg[g][h] - mx[g][h]) for h in range(HEADS)] for g in gs]
    att = [[(e[g][h] / jnp.sum(e[g][h], axis=1, keepdims=True)).astype(_BF)
            for h in range(HEADS)] for g in gs]
    oh = [[jnp.dot(att[g][h], vh[g][h].astype(_BF), preferred_element_type=_F)
           for h in range(HEADS)] for g in gs]
    attn = [bout_ref[...] * jnp.ones((M_Q + K_TOP, 1), _F) for g in gs]
    for h in range(HEADS):
        attn = [attn[g] + jnp.dot(oh[g][h].astype(_BF), woh_ref[h],
                                  preferred_element_type=_F) for g in gs]
    enh = [comb[g] + attn[g] for g in gs]
    delta = [(jnp.dot(enh[g][M_Q:, :].astype(_BF), wup_ref[...],
                      preferred_element_type=_F) * gam).astype(_BF) for g in gs]
    scat = [lax.dot_general(pmat[g], delta[g], (((0,), (0,)), ((), ())),
                            preferred_element_type=_F) for g in gs]
    for g in gs:
        o_ref[g] = x_ref[g] + gbu + scat[g]


def kernel(image_features, text_features, W_down, b_down, W_omni, b_omni,
           W_up, b_up, m_queries, W_in, b_in, W_out, b_out, ln_w, ln_b, gamma):
    B, N, C = image_features.shape
    D = W_omni.shape[0]
    T = W_down.shape[0]
    pooled = text_features[:, None, 0, :]
    wd = W_down.T.astype(_BF)
    wo = W_omni.T.astype(_BF)
    bd = b_down.reshape(1, T)
    bo = b_omni.reshape(1, D)

    scores = pl.pallas_call(
        _k1,
        grid=(B // _KB,),
        in_specs=[
            pl.BlockSpec((_KB, N, C), lambda b: (b, 0, 0)),
            pl.BlockSpec((_KB, 1, T), lambda b: (b, 0, 0)),
            pl.BlockSpec((C, T), lambda b: (0, 0)),
            pl.BlockSpec((T, D), lambda b: (0, 0)),
            pl.BlockSpec((1, T), lambda b: (0, 0)),
            pl.BlockSpec((1, D), lambda b: (0, 0)),
        ],
        out_specs=pl.BlockSpec((_KB, 1, N), lambda b: (b, 0, 0)),
        out_shape=jax.ShapeDtypeStruct((B, 1, N), _F),
        interpret=False,
    )(image_features, pooled, wd, wo, bd, bo)

    sel = _sc_topk_build(B, N)(scores.reshape(B * N))
    sel3 = sel.reshape(B, 1, N)

    wq = W_in[:D].T
    wk = W_in[D:2 * D].T
    wv = W_in[2 * D:].T
    wh = jnp.stack(
        [wq[:, h * HEAD_DIM:(h + 1) * HEAD_DIM] for h in range(HEADS)]
        + [wk[:, h * HEAD_DIM:(h + 1) * HEAD_DIM] for h in range(HEADS)]
        + [wv[:, h * HEAD_DIM:(h + 1) * HEAD_DIM] for h in range(HEADS)]
    ).astype(_BF)
    bh = jnp.stack([b_in[i * HEAD_DIM:(i + 1) * HEAD_DIM].reshape(1, HEAD_DIM)
                    for i in range(3 * HEADS)])
    wot = W_out.T
    woh = jnp.stack([wot[h * HEAD_DIM:(h + 1) * HEAD_DIM, :]
                     for h in range(HEADS)]).astype(_BF)
    mq = m_queries[0]
    lnw2 = ln_w.reshape(1, D)
    lnb2 = ln_b.reshape(1, D)
    bout2 = b_out.reshape(1, D)
    wup = W_up.T.astype(_BF)
    bu2 = b_up.reshape(1, C)
    g2 = jnp.reshape(gamma, (1, 1)).astype(_F)

    out = pl.pallas_call(
        _k34,
        grid=(B // _G,),
        in_specs=[
            pl.BlockSpec((_G, 1, N), lambda b: (b, 0, 0)),
            pl.BlockSpec((_G, N, C), lambda b: (b, 0, 0)),
            pl.BlockSpec((C, T), lambda b: (0, 0)),
            pl.BlockSpec((T, D), lambda b: (0, 0)),
            pl.BlockSpec((1, T), lambda b: (0, 0)),
            pl.BlockSpec((1, D), lambda b: (0, 0)),
            pl.BlockSpec((M_Q, D), lambda b: (0, 0)),
            pl.BlockSpec((1, D), lambda b: (0, 0)),
            pl.BlockSpec((1, D), lambda b: (0, 0)),
            pl.BlockSpec((3 * HEADS, D, HEAD_DIM), lambda b: (0, 0, 0)),
            pl.BlockSpec((3 * HEADS, 1, HEAD_DIM), lambda b: (0, 0, 0)),
            pl.BlockSpec((HEADS, HEAD_DIM, D), lambda b: (0, 0, 0)),
            pl.BlockSpec((1, D), lambda b: (0, 0)),
            pl.BlockSpec((D, C), lambda b: (0, 0)),
            pl.BlockSpec((1, C), lambda b: (0, 0)),
            pl.BlockSpec((1, 1), lambda b: (0, 0)),
        ],
        out_specs=pl.BlockSpec((_G, N, C), lambda b: (b, 0, 0)),
        out_shape=jax.ShapeDtypeStruct((B, N, C), _F),
        interpret=False,
    )(sel3, image_features, wd, wo, bd, bo, mq, lnw2, lnb2, wh, bh, woh,
      bout2, wup, bu2, g2)
    return out


# speedup vs baseline: 1.2137x; 1.0017x over previous
import functools

import jax
import jax.numpy as jnp
from jax import lax
from jax.experimental import pallas as pl
from jax.experimental.pallas import tpu as pltpu
from jax.experimental.pallas import tpu_sc as plsc

_BF = jnp.bfloat16
_F = jnp.float32

K_TOP = 64
M_Q = 16
HEADS = 4
HEAD_DIM = 16
_KB = 4
_G = 8


def _gelu(x):
    return 0.5 * x * (1.0 + lax.erf(x * 0.7071067811865476))


def _k1(x_ref, pooled_ref, wd_ref, wo_ref, bd_ref, bo_ref, sc_ref):
    kb, n = x_ref.shape[0], x_ref.shape[1]
    x = x_ref[...].reshape(kb * n, -1).astype(_BF)
    proj = jnp.dot(x, wd_ref[...], preferred_element_type=_F) + bd_ref[...]
    proj = _gelu(proj)
    act = jnp.dot(proj.astype(_BF), wo_ref[...], preferred_element_type=_F) + bo_ref[...]
    ptxt = jnp.dot(pooled_ref[:, 0, :].astype(_BF), wo_ref[...],
                   preferred_element_type=_F) + bo_ref[...]
    w = (ptxt + 1e-8).astype(_BF)
    a2 = (act + 1e-8).astype(_BF)
    ones = jnp.ones((1,) + w.shape[1:], _BF)
    for i in range(kb):
        a2i = a2[i * n:(i + 1) * n, :]
        num = lax.dot_general(w[i:i + 1, :], a2i, (((1,), (1,)), ((), ())),
                              preferred_element_type=_F)
        nrm2 = lax.dot_general(ones, a2i * a2i, (((1,), (1,)), ((), ())),
                               preferred_element_type=_F)
        sc_ref[i] = num / jnp.maximum(jnp.sqrt(nrm2), 1e-12)


def _sc_topk_build(B, N):
    mesh = plsc.VectorSubcoreMesh(core_axis_name="c", subcore_axis_name="s")
    nchunk = N // 16

    @functools.partial(
        pl.kernel,
        out_type=jax.ShapeDtypeStruct((B * N,), jnp.int32),
        mesh=mesh,
        scratch_types=[
            pltpu.VMEM((N,), _F),
            pltpu.VMEM((N,), jnp.uint32),
            pltpu.VMEM((N,), jnp.int32),
        ],
        compiler_params=pltpu.CompilerParams(needs_layout_passes=False),
    )
    def sc_topk(sc_hbm, sel_hbm, s_v, m_v, sel_v):
        wid = lax.axis_index("s") * 2 + lax.axis_index("c")
        base = wid * N
        pltpu.sync_copy(sc_hbm.at[pl.ds(base, N)], s_v)

        def mapb(i, carry):
            u = plsc.bitcast(s_v[pl.ds(i * 16, 16)], jnp.uint32)
            neg = u >= jnp.uint32(0x80000000)
            m_v[pl.ds(i * 16, 16)] = jnp.where(
                neg, u ^ jnp.uint32(0xFFFFFFFF), u | jnp.uint32(0x80000000))
            return carry

        lax.fori_loop(0, nchunk, mapb, 0)

        def count_ge(th):
            def cb(i, acc):
                ge = m_v[pl.ds(i * 16, 16)] >= th
                return acc + plsc.all_reduce_population_count(ge)

            return lax.fori_loop(0, nchunk, cb, jnp.zeros((16,), jnp.int32))

        one = jnp.ones((16,), jnp.uint32)

        def bitb(j, acc):
            cand = acc | (one << (31 - j))
            return jnp.where(count_ge(cand) >= K_TOP, cand, acc)

        t = lax.fori_loop(0, 32, bitb, jnp.zeros((16,), jnp.uint32))
        budget = K_TOP - count_ge(t + 1)

        def comp(i, carry):
            nsel, tused = carry
            m = m_v[pl.ds(i * 16, 16)]
            gt = m > t
            eq = m == t
            ceq = plsc.cumsum(eq.astype(jnp.int32))
            take_eq = eq & ((tused + ceq) <= budget)
            selm = gt | take_eq
            csel = plsc.cumsum(selm.astype(jnp.int32))
            sel_v[pl.ds(i * 16, 16)] = jnp.where(selm, nsel + csel - 1, -1)
            return (nsel + plsc.all_reduce_population_count(selm),
                    tused + plsc.all_reduce_population_count(take_eq))

        z = jnp.zeros((16,), jnp.int32)
        lax.fori_loop(0, nchunk, comp, (z, z))
        pltpu.sync_copy(sel_v, sel_hbm.at[pl.ds(base, N)])

    return sc_topk


def _k34(sel_ref, x_ref, wd_ref, wo_ref, bd_ref, bo_ref, mq_ref, lnw_ref,
         lnb_ref, wh_ref, bh_ref, woh_ref, bout_ref, wup_ref, bu_ref, g_ref,
         o_ref):
    gam = g_ref[0, 0]
    gbu = gam * bu_ref[...]
    gs = range(_G)
    n = sel_ref.shape[2]
    kio = lax.broadcasted_iota(jnp.int32, (K_TOP, n), 0)
    pmat = [(jnp.broadcast_to(sel_ref[g], (K_TOP, n)) == kio).astype(_BF)
            for g in gs]
    xg = [jnp.dot(pmat[g], x_ref[g].astype(_BF), preferred_element_type=_F)
          for g in gs]
    pj = [_gelu(jnp.dot(xg[g].astype(_BF), wd_ref[...],
                        preferred_element_type=_F) + bd_ref[...]) for g in gs]
    sparse = [jnp.dot(pj[g].astype(_BF), wo_ref[...],
                      preferred_element_type=_F) + bo_ref[...] for g in gs]
    comb = [jnp.concatenate([mq_ref[...], sparse[g]], axis=0) for g in gs]
    mu = [jnp.mean(comb[g], axis=1, keepdims=True) for g in gs]
    var = [jnp.mean((comb[g] - mu[g]) ** 2, axis=1, keepdims=True) for g in gs]
    ln16 = [((comb[g] - mu[g]) * lax.rsqrt(var[g] + 1e-5) * lnw_ref[...]
             + lnb_ref[...]).astype(_BF) for g in gs]
    qh = [[jnp.dot(ln16[g], wh_ref[h], preferred_element_type=_F) + bh_ref[h]
           for h in range(HEADS)] for g in gs]
    kh = [[jnp.dot(ln16[g], wh_ref[HEADS + h], preferred_element_type=_F)
           + bh_ref[HEADS + h] for h in range(HEADS)] for g in gs]
    vh = [[jnp.dot(ln16[g], wh_ref[2 * HEADS + h], preferred_element_type=_F)
           + bh_ref[2 * HEADS + h] for h in range(HEADS)] for g in gs]
    lg = [[lax.dot_general(qh[g][h].astype(_BF), kh[g][h].astype(_BF),
                           (((1,), (1,)), ((), ())),
                           preferred_element_type=_F) * (1.0 / (HEAD_DIM ** 0.5))
           for h in range(HEADS)] for g in gs]
    mx = [[jnp.max(lg[g][h], axis=1, keepdims=True) for h in range(HEADS)]
          for g in gs]
    e = [[jnp.exp(lg[g][h] - mx[g][h]) for h in range(HEADS)] for g in gs]
    att = [[(e[g][h] / jnp.sum(e[g][h], axis=1, keepdims=True)).astype(_BF)
            for h in range(HEADS)] for g in gs]
    oh = [[jnp.dot(att[g][h], vh[g][h].astype(_BF), preferred_element_type=_F)
           for h in range(HEADS)] for g in gs]
    attn = [bout_ref[...] * jnp.ones((M_Q + K_TOP, 1), _F) for g in gs]
    for h in range(HEADS):
        attn = [attn[g] + jnp.dot(oh[g][h].astype(_BF), woh_ref[h],
                                  preferred_element_type=_F) for g in gs]
    enh = [comb[g] + attn[g] for g in gs]
    delta = [(jnp.dot(enh[g][M_Q:, :].astype(_BF), wup_ref[...],
                      preferred_element_type=_F) * gam).astype(_BF) for g in gs]
    scat = [lax.dot_general(pmat[g], delta[g], (((0,), (0,)), ((), ())),
                            preferred_element_type=_F) for g in gs]
    for g in gs:
        o_ref[g] = x_ref[g] + gbu + scat[g]


def kernel(image_features, text_features, W_down, b_down, W_omni, b_omni,
           W_up, b_up, m_queries, W_in, b_in, W_out, b_out, ln_w, ln_b, gamma):
    B, N, C = image_features.shape
    D = W_omni.shape[0]
    T = W_down.shape[0]
    pooled = text_features[:, None, 0, :]
    wd = W_down.T.astype(_BF)
    wo = W_omni.T.astype(_BF)
    bd = b_down.reshape(1, T)
    bo = b_omni.reshape(1, D)

    scores = pl.pallas_call(
        _k1,
        grid=(B // _KB,),
        in_specs=[
            pl.BlockSpec((_KB, N, C), lambda b: (b, 0, 0)),
            pl.BlockSpec((_KB, 1, T), lambda b: (b, 0, 0)),
            pl.BlockSpec((C, T), lambda b: (0, 0)),
            pl.BlockSpec((T, D), lambda b: (0, 0)),
            pl.BlockSpec((1, T), lambda b: (0, 0)),
            pl.BlockSpec((1, D), lambda b: (0, 0)),
        ],
        out_specs=pl.BlockSpec((_KB, 1, N), lambda b: (b, 0, 0)),
        out_shape=jax.ShapeDtypeStruct((B, 1, N), _F),
        interpret=False,
    )(image_features, pooled, wd, wo, bd, bo)

    sel = _sc_topk_build(B, N)(scores.reshape(B * N))
    sel3 = sel.reshape(B, 1, N)

    wq = W_in[:D].T
    wk = W_in[D:2 * D].T
    wv = W_in[2 * D:].T
    wh = jnp.stack(
        [wq[:, h * HEAD_DIM:(h + 1) * HEAD_DIM] for h in range(HEADS)]
        + [wk[:, h * HEAD_DIM:(h + 1) * HEAD_DIM] for h in range(HEADS)]
        + [wv[:, h * HEAD_DIM:(h + 1) * HEAD_DIM] for h in range(HEADS)]
    ).astype(_BF)
    bh = jnp.stack([b_in[i * HEAD_DIM:(i + 1) * HEAD_DIM].reshape(1, HEAD_DIM)
                    for i in range(3 * HEADS)])
    wot = W_out.T
    woh = jnp.stack([wot[h * HEAD_DIM:(h + 1) * HEAD_DIM, :]
                     for h in range(HEADS)]).astype(_BF)
    mq = m_queries[0]
    lnw2 = ln_w.reshape(1, D)
    lnb2 = ln_b.reshape(1, D)
    bout2 = b_out.reshape(1, D)
    wup = W_up.T.astype(_BF)
    bu2 = b_up.reshape(1, C)
    g2 = jnp.reshape(gamma, (1, 1)).astype(_F)

    out = pl.pallas_call(
        _k34,
        grid=(B // _G,),
        in_specs=[
            pl.BlockSpec((_G, 1, N), lambda b: (b, 0, 0)),
            pl.BlockSpec((_G, N, C), lambda b: (b, 0, 0)),
            pl.BlockSpec((C, T), lambda b: (0, 0)),
            pl.BlockSpec((T, D), lambda b: (0, 0)),
            pl.BlockSpec((1, T), lambda b: (0, 0)),
            pl.BlockSpec((1, D), lambda b: (0, 0)),
            pl.BlockSpec((M_Q, D), lambda b: (0, 0)),
            pl.BlockSpec((1, D), lambda b: (0, 0)),
            pl.BlockSpec((1, D), lambda b: (0, 0)),
            pl.BlockSpec((3 * HEADS, D, HEAD_DIM), lambda b: (0, 0, 0)),
            pl.BlockSpec((3 * HEADS, 1, HEAD_DIM), lambda b: (0, 0, 0)),
            pl.BlockSpec((HEADS, HEAD_DIM, D), lambda b: (0, 0, 0)),
            pl.BlockSpec((1, D), lambda b: (0, 0)),
            pl.BlockSpec((D, C), lambda b: (0, 0)),
            pl.BlockSpec((1, C), lambda b: (0, 0)),
            pl.BlockSpec((1, 1), lambda b: (0, 0)),
        ],
        out_specs=pl.BlockSpec((_G, N, C), lambda b: (b, 0, 0)),
        out_shape=jax.ShapeDtypeStruct((B, N, C), _F),
        interpret=False,
    )(sel3, image_features, wd, wo, bd, bo, mq, lnw2, lnb2, wh, bh, woh,
      bout2, wup, bu2, g2)
    return out
